# baseline (speedup 1.0000x reference)
import functools
import numpy as np
import jax
import jax.numpy as jnp
from jax.experimental import pallas as pl
from jax.experimental.pallas import tpu as pltpu

NUM_RBF = 16
NUM_PE = 16
TOP_K = 30
N_ATOMS = 14
NPAIR = N_ATOMS * N_ATOMS
LQ = 64


def _body(xcaT_ref, xperm_ref, xhi_ref, xlo_ref, a_ref, freq_ref, wpe_ref,
          wrbf_ref, g_ref, bt_ref, ekm_ref, eidx_ref):
    t = pl.program_id(1)
    base = t * LQ

    d2 = None
    for c in range(3):
        xall = xcaT_ref[0, c:c + 1, :]
        xq = xperm_ref[0, c, pl.ds(base, LQ), 1:2]
        diff = xq - xall
        d2 = diff * diff if d2 is None else d2 + diff * diff
    dca = jnp.sqrt(d2 + 1e-6)

    lane512 = jax.lax.broadcasted_iota(jnp.int32, (LQ, 512), 1)
    lvals = (base + jax.lax.broadcasted_iota(jnp.int32, (LQ, 1), 0)
             ).astype(jnp.float32)
    freq_row = freq_ref[0:1, :]

    sel_cols = []
    oh_blocks = []
    dpos_blocks = []
    work = dca
    for _ in range(TOP_K):
        m = jnp.min(work, axis=1, keepdims=True)
        cand = jnp.where(work == m, lane512, 512)
        sel = jnp.min(cand, axis=1, keepdims=True)
        hit = lane512 == sel
        work = jnp.where(hit, jnp.float32(np.inf), work)
        sel_cols.append(sel)
        oh_blocks.append(hit.astype(jnp.bfloat16))
        dpos_blocks.append(sel.astype(jnp.float32) - lvals)

    eidx_ref[0] = jnp.concatenate(sel_cols, axis=1)

    dcol = jnp.concatenate(dpos_blocks, axis=0)
    ang = dcol * freq_row
    lane16r = jax.lax.broadcasted_iota(jnp.int32, (LQ * TOP_K, NUM_PE), 1)
    epos = jnp.where(lane16r < 8, jnp.cos(ang), jnp.sin(ang))

    oh = jnp.concatenate(oh_blocks, axis=0)
    xn48 = (jnp.dot(oh, xhi_ref[0], preferred_element_type=jnp.float32)
            + jnp.dot(oh, xlo_ref[0], preferred_element_type=jnp.float32))

    hp = jax.lax.Precision.HIGHEST
    d2nb = None
    for c in range(3):
        qc = xperm_ref[0, c, pl.ds(base, LQ), :]
        xn = xn48[:, c * 16:c * 16 + N_ATOMS]
        nexp = jnp.concatenate([xn] * N_ATOMS, axis=1)
        qa = jnp.dot(qc, a_ref[...], preferred_element_type=jnp.float32,
                     precision=hp)
        qexp = jnp.concatenate([qa] * TOP_K, axis=0)
        diff = qexp - nexp
        d2nb = diff * diff if d2nb is None else d2nb + diff * diff
    dnb = jnp.sqrt(d2nb + 1e-6)

    acc = jnp.dot(epos, wpe_ref[...], preferred_element_type=jnp.float32)
    mus = np.linspace(0.0, 20.0, NUM_RBF).astype(np.float32)
    inv_sigma = np.float32(NUM_RBF / 20.0)
    u = dnb * inv_sigma
    w = -(u * u)
    for mi in range(NUM_RBF):
        cm = np.float32(mus[mi] * inv_sigma)
        g = jnp.exp(u * np.float32(2.0 * cm) + (w - np.float32(cm * cm)))
        acc = acc + jnp.dot(g, wrbf_ref[mi],
                            preferred_element_type=jnp.float32)

    mu = jnp.mean(acc, axis=1, keepdims=True)
    xc_ = acc - mu
    var = jnp.mean(xc_ * xc_, axis=1, keepdims=True)
    y = (xc_ * jax.lax.rsqrt(var + 1e-5)) * g_ref[0:1, :] + bt_ref[0:1, :]

    ekm_ref[0] = y.reshape(TOP_K, LQ, 128)


@jax.jit
def _run(xcaT, xperm, xhi, xlo, a_m, freq8, wpe, wrbf, gamma, beta):
    nb, _, nl = xcaT.shape
    grid = (nb, nl // LQ)
    ekm, eidx = pl.pallas_call(
        _body,
        grid=grid,
        in_specs=[
            pl.BlockSpec((1, 3, nl), lambda b, t: (b, 0, 0)),
            pl.BlockSpec((1, 3, nl, N_ATOMS), lambda b, t: (b, 0, 0, 0)),
            pl.BlockSpec((1, nl, 48), lambda b, t: (b, 0, 0)),
            pl.BlockSpec((1, nl, 48), lambda b, t: (b, 0, 0)),
            pl.BlockSpec((N_ATOMS, NPAIR), lambda b, t: (0, 0)),
            pl.BlockSpec((1, NUM_PE), lambda b, t: (0, 0)),
            pl.BlockSpec((NUM_PE, 128), lambda b, t: (0, 0)),
            pl.BlockSpec((NUM_RBF, NPAIR, 128), lambda b, t: (0, 0, 0)),
            pl.BlockSpec((1, 128), lambda b, t: (0, 0)),
            pl.BlockSpec((1, 128), lambda b, t: (0, 0)),
        ],
        out_specs=[
            pl.BlockSpec((1, TOP_K, LQ, 128), lambda b, t: (b, 0, t, 0)),
            pl.BlockSpec((1, LQ, TOP_K), lambda b, t: (b, t, 0)),
        ],
        out_shape=[
            jax.ShapeDtypeStruct((nb, TOP_K, nl, 128), jnp.float32),
            jax.ShapeDtypeStruct((nb, nl, TOP_K), jnp.int32),
        ],
        compiler_params=pltpu.CompilerParams(
            dimension_semantics=("parallel", "arbitrary"),
        ),
    )(xcaT, xperm, xhi, xlo, a_m, freq8, wpe, wrbf, gamma, beta)
    e_out = jnp.transpose(ekm, (0, 2, 1, 3))
    return e_out, eidx


def kernel(X, mask, atom_mask, W_e, ln_gamma, ln_beta):
    xperm = jnp.transpose(X, (0, 3, 1, 2))
    xcaT = xperm[:, :, :, 1]

    xt = jnp.transpose(X, (0, 1, 3, 2))
    xt = jnp.pad(xt, ((0, 0), (0, 0), (0, 0), (0, 2)))
    x48 = xt.reshape(X.shape[0], X.shape[1], 48)
    xhi = x48.astype(jnp.bfloat16)
    xlo = (x48 - xhi.astype(jnp.float32)).astype(jnp.bfloat16)

    p = np.arange(NPAIR)
    a_m = jnp.asarray((p[None, :] // N_ATOMS
                       == np.arange(N_ATOMS)[:, None]).astype(np.float32))

    freq = jnp.exp(jnp.arange(0, NUM_PE, 2, dtype=jnp.float32)
                   * (-(np.log(10000.0) / NUM_PE)))
    freq8 = jnp.concatenate([freq, freq]).reshape(1, NUM_PE)

    wpe = W_e[:, :NUM_PE].T
    wrbf = (W_e[:, NUM_PE:].T
            .reshape(NPAIR, NUM_RBF, 128)
            .transpose(1, 0, 2))
    gamma = ln_gamma.reshape(1, 128)
    beta = ln_beta.reshape(1, 128)
    return _run(xcaT, xperm, xhi, xlo, a_m, freq8, wpe, wrbf, gamma, beta)


# speedup vs baseline: 1.0514x; 1.0514x over previous
import functools
import numpy as np
import jax
import jax.numpy as jnp
from jax.experimental import pallas as pl
from jax.experimental.pallas import tpu as pltpu

NUM_RBF = 16
NUM_PE = 16
TOP_K = 30
N_ATOMS = 14
NPAIR = N_ATOMS * N_ATOMS
LQ = 64


def _body(xcaT_ref, xperm_ref, xhi_ref, xlo_ref, a_ref, freq_ref, shift_ref,
          wpe_ref, wrbf_ref, g_ref, bt_ref, ekm_ref, eidx_ref):
    t = pl.program_id(1)
    base = t * LQ

    d2 = None
    for c in range(3):
        xall = xcaT_ref[0, c:c + 1, :]
        xq = xperm_ref[0, c, pl.ds(base, LQ), 1:2]
        diff = xq - xall
        d2 = diff * diff if d2 is None else d2 + diff * diff
    dca = jnp.sqrt(d2 + 1e-6)

    lane512 = jax.lax.broadcasted_iota(jnp.int32, (LQ, 512), 1)
    lvals = (base + jax.lax.broadcasted_iota(jnp.int32, (LQ, 1), 0)
             ).astype(jnp.float32)
    freq_row = freq_ref[0:1, :]

    sel_cols = []
    oh_blocks = []
    dpos_blocks = []
    work = dca
    for _ in range(TOP_K):
        m = jnp.min(work, axis=1, keepdims=True)
        cand = jnp.where(work == m, lane512, 512)
        sel = jnp.min(cand, axis=1, keepdims=True)
        hit = lane512 == sel
        work = jnp.where(hit, jnp.float32(np.inf), work)
        sel_cols.append(sel)
        oh_blocks.append(hit.astype(jnp.float32))
        dpos_blocks.append(sel.astype(jnp.float32) - lvals)

    eidx_ref[0] = jnp.concatenate(sel_cols, axis=1)

    dcol = jnp.concatenate(dpos_blocks, axis=0)
    epos = jnp.cos(dcol * freq_row - shift_ref[0:1, :])

    oh = jnp.concatenate(oh_blocks, axis=0)
    xn48 = (jnp.dot(oh, xhi_ref[0], preferred_element_type=jnp.float32)
            + jnp.dot(oh, xlo_ref[0], preferred_element_type=jnp.float32))

    hp = jax.lax.Precision.HIGHEST
    d2nb = None
    for c in range(3):
        qc = xperm_ref[0, c, pl.ds(base, LQ), :]
        xn = xn48[:, c * 16:c * 16 + N_ATOMS]
        nexp = jnp.concatenate([xn] * N_ATOMS, axis=1)
        qa = jnp.dot(qc, a_ref[...], preferred_element_type=jnp.float32,
                     precision=hp)
        qexp = jnp.concatenate([qa] * TOP_K, axis=0)
        diff = qexp - nexp
        d2nb = diff * diff if d2nb is None else d2nb + diff * diff
    dnb = jnp.sqrt(d2nb + 1e-6)

    acc = jnp.dot(epos, wpe_ref[...], preferred_element_type=jnp.float32)
    mus = np.linspace(0.0, 20.0, NUM_RBF).astype(np.float32)
    inv_sigma = np.float32(NUM_RBF / 20.0)
    u = dnb * inv_sigma
    w = -(u * u)
    for mi in range(NUM_RBF):
        cm = np.float32(mus[mi] * inv_sigma)
        g = jnp.exp(u * np.float32(2.0 * cm) + (w - np.float32(cm * cm)))
        acc = acc + jnp.dot(g, wrbf_ref[mi],
                            preferred_element_type=jnp.float32)

    mu = jnp.mean(acc, axis=1, keepdims=True)
    xc_ = acc - mu
    var = jnp.mean(xc_ * xc_, axis=1, keepdims=True)
    y = (xc_ * jax.lax.rsqrt(var + 1e-5)) * g_ref[0:1, :] + bt_ref[0:1, :]

    ekm_ref[0] = y.reshape(TOP_K, LQ, 128)


@jax.jit
def _run(xcaT, xperm, xhi, xlo, a_m, freq8, shift16, wpe, wrbf, gamma, beta):
    nb, _, nl = xcaT.shape
    grid = (nb, nl // LQ)
    ekm, eidx = pl.pallas_call(
        _body,
        grid=grid,
        in_specs=[
            pl.BlockSpec((1, 3, nl), lambda b, t: (b, 0, 0)),
            pl.BlockSpec((1, 3, nl, N_ATOMS), lambda b, t: (b, 0, 0, 0)),
            pl.BlockSpec((1, nl, 48), lambda b, t: (b, 0, 0)),
            pl.BlockSpec((1, nl, 48), lambda b, t: (b, 0, 0)),
            pl.BlockSpec((N_ATOMS, NPAIR), lambda b, t: (0, 0)),
            pl.BlockSpec((1, NUM_PE), lambda b, t: (0, 0)),
            pl.BlockSpec((1, NUM_PE), lambda b, t: (0, 0)),
            pl.BlockSpec((NUM_PE, 128), lambda b, t: (0, 0)),
            pl.BlockSpec((NUM_RBF, NPAIR, 128), lambda b, t: (0, 0, 0)),
            pl.BlockSpec((1, 128), lambda b, t: (0, 0)),
            pl.BlockSpec((1, 128), lambda b, t: (0, 0)),
        ],
        out_specs=[
            pl.BlockSpec((1, TOP_K, LQ, 128), lambda b, t: (b, 0, t, 0)),
            pl.BlockSpec((1, LQ, TOP_K), lambda b, t: (b, t, 0)),
        ],
        out_shape=[
            jax.ShapeDtypeStruct((nb, TOP_K, nl, 128), jnp.float32),
            jax.ShapeDtypeStruct((nb, nl, TOP_K), jnp.int32),
        ],
        compiler_params=pltpu.CompilerParams(
            dimension_semantics=("parallel", "arbitrary"),
        ),
    )(xcaT, xperm, xhi, xlo, a_m, freq8, shift16, wpe, wrbf, gamma, beta)
    e_out = jnp.transpose(ekm, (0, 2, 1, 3))
    return e_out, eidx


def kernel(X, mask, atom_mask, W_e, ln_gamma, ln_beta):
    xperm = jnp.transpose(X, (0, 3, 1, 2))
    xcaT = xperm[:, :, :, 1]

    xt = jnp.transpose(X, (0, 1, 3, 2))
    xt = jnp.pad(xt, ((0, 0), (0, 0), (0, 0), (0, 2)))
    x48 = xt.reshape(X.shape[0], X.shape[1], 48)
    xhi = x48.astype(jnp.bfloat16)
    xlo = (x48 - xhi.astype(jnp.float32)).astype(jnp.bfloat16)

    p = np.arange(NPAIR)
    a_m = jnp.asarray((p[None, :] // N_ATOMS
                       == np.arange(N_ATOMS)[:, None]).astype(np.float32))

    freq = jnp.exp(jnp.arange(0, NUM_PE, 2, dtype=jnp.float32)
                   * (-(np.log(10000.0) / NUM_PE)))
    freq8 = jnp.concatenate([freq, freq]).reshape(1, NUM_PE)
    shift16 = jnp.concatenate([jnp.zeros((8,), jnp.float32),
                               jnp.full((8,), np.float32(np.pi / 2))]
                              ).reshape(1, NUM_PE)

    wpe = W_e[:, :NUM_PE].T
    wrbf = (W_e[:, NUM_PE:].T
            .reshape(NPAIR, NUM_RBF, 128)
            .transpose(1, 0, 2))
    gamma = ln_gamma.reshape(1, 128)
    beta = ln_beta.reshape(1, 128)
    return _run(xcaT, xperm, xhi, xlo, a_m, freq8, shift16, wpe, wrbf,
                gamma, beta)


# speedup vs baseline: 1.0959x; 1.0423x over previous
import functools
import numpy as np
import jax
import jax.numpy as jnp
from jax.experimental import pallas as pl
from jax.experimental.pallas import tpu as pltpu

NUM_RBF = 16
NUM_PE = 16
TOP_K = 30
N_ATOMS = 14
NPAIR = N_ATOMS * N_ATOMS
LQ = 64


def _body(xcaT_ref, xperm_ref, xhi_ref, xlo_ref, a_ref, uv_ref,
          wpe_ref, wrbf_ref, g_ref, bt_ref, ekm_ref, eidx_ref, oh_scr):
    t = pl.program_id(1)
    base = t * LQ

    d2 = None
    for c in range(3):
        xall = xcaT_ref[0, c:c + 1, :]
        xq = xperm_ref[0, c, pl.ds(base, LQ), 1:2]
        diff = xq - xall
        d2 = diff * diff if d2 is None else d2 + diff * diff
    dca = jnp.sqrt(d2 + 1e-6)

    lane512 = jax.lax.broadcasted_iota(jnp.int32, (LQ, 512), 1)

    sel_cols = []
    work = dca
    for k in range(TOP_K):
        m = jnp.min(work, axis=1, keepdims=True)
        cand = jnp.where(work == m, lane512, 512)
        sel = jnp.min(cand, axis=1, keepdims=True)
        hit = lane512 == sel
        work = jnp.where(hit, jnp.float32(np.inf), work)
        sel_cols.append(sel)
        oh_scr[pl.ds(k * LQ, LQ), :] = hit.astype(jnp.float32)

    eidx_ref[0] = jnp.concatenate(sel_cols, axis=1)

    oh = oh_scr[...]
    xn80 = (jnp.dot(oh, xhi_ref[0], preferred_element_type=jnp.float32)
            + jnp.dot(oh, xlo_ref[0], preferred_element_type=jnp.float32))

    urep = jnp.concatenate([uv_ref[:, :NUM_PE]] * TOP_K, axis=0)
    vrep = jnp.concatenate([uv_ref[:, NUM_PE:]] * TOP_K, axis=0)
    epos = xn80[:, 48:64] * urep + xn80[:, 64:80] * vrep

    hp = jax.lax.Precision.HIGHEST
    d2nb = None
    for c in range(3):
        qc = xperm_ref[0, c, pl.ds(base, LQ), :]
        xn = xn80[:, c * 16:c * 16 + N_ATOMS]
        nexp = jnp.concatenate([xn] * N_ATOMS, axis=1)
        qa = jnp.dot(qc, a_ref[...], preferred_element_type=jnp.float32,
                     precision=hp)
        qexp = jnp.concatenate([qa] * TOP_K, axis=0)
        diff = qexp - nexp
        d2nb = diff * diff if d2nb is None else d2nb + diff * diff
    dnb = jnp.sqrt(d2nb + 1e-6)

    acc = jnp.dot(epos, wpe_ref[...], preferred_element_type=jnp.float32)
    mus = np.linspace(0.0, 20.0, NUM_RBF).astype(np.float32)
    inv_sigma = np.float32(NUM_RBF / 20.0)
    u = dnb * inv_sigma
    w = -(u * u)
    for mi in range(NUM_RBF):
        cm = np.float32(mus[mi] * inv_sigma)
        g = jnp.exp(u * np.float32(2.0 * cm) + (w - np.float32(cm * cm)))
        acc = acc + jnp.dot(g, wrbf_ref[mi],
                            preferred_element_type=jnp.float32)

    mu = jnp.mean(acc, axis=1, keepdims=True)
    xc_ = acc - mu
    var = jnp.mean(xc_ * xc_, axis=1, keepdims=True)
    y = (xc_ * jax.lax.rsqrt(var + 1e-5)) * g_ref[0:1, :] + bt_ref[0:1, :]

    ekm_ref[0] = y.reshape(TOP_K, LQ, 128)


@jax.jit
def _run(xcaT, xperm, xhi, xlo, a_m, uv, wpe, wrbf, gamma, beta):
    nb, _, nl = xcaT.shape
    grid = (nb, nl // LQ)
    ekm, eidx = pl.pallas_call(
        _body,
        grid=grid,
        in_specs=[
            pl.BlockSpec((1, 3, nl), lambda b, t: (b, 0, 0)),
            pl.BlockSpec((1, 3, nl, N_ATOMS), lambda b, t: (b, 0, 0, 0)),
            pl.BlockSpec((1, nl, 80), lambda b, t: (b, 0, 0)),
            pl.BlockSpec((1, nl, 80), lambda b, t: (b, 0, 0)),
            pl.BlockSpec((N_ATOMS, NPAIR), lambda b, t: (0, 0)),
            pl.BlockSpec((LQ, 2 * NUM_PE), lambda b, t: (t, 0)),
            pl.BlockSpec((NUM_PE, 128), lambda b, t: (0, 0)),
            pl.BlockSpec((NUM_RBF, NPAIR, 128), lambda b, t: (0, 0, 0)),
            pl.BlockSpec((1, 128), lambda b, t: (0, 0)),
            pl.BlockSpec((1, 128), lambda b, t: (0, 0)),
        ],
        scratch_shapes=[pltpu.VMEM((LQ * TOP_K, 512), jnp.float32)],
        out_specs=[
            pl.BlockSpec((1, TOP_K, LQ, 128), lambda b, t: (b, 0, t, 0)),
            pl.BlockSpec((1, LQ, TOP_K), lambda b, t: (b, t, 0)),
        ],
        out_shape=[
            jax.ShapeDtypeStruct((nb, TOP_K, nl, 128), jnp.float32),
            jax.ShapeDtypeStruct((nb, nl, TOP_K), jnp.int32),
        ],
        compiler_params=pltpu.CompilerParams(
            dimension_semantics=("parallel", "arbitrary"),
        ),
    )(xcaT, xperm, xhi, xlo, a_m, uv, wpe, wrbf, gamma, beta)
    e_out = jnp.transpose(ekm, (0, 2, 1, 3))
    return e_out, eidx


def kernel(X, mask, atom_mask, W_e, ln_gamma, ln_beta):
    xperm = jnp.transpose(X, (0, 3, 1, 2))
    xcaT = xperm[:, :, :, 1]

    nb, nl = X.shape[0], X.shape[1]
    xt = jnp.transpose(X, (0, 1, 3, 2))
    xt = jnp.pad(xt, ((0, 0), (0, 0), (0, 0), (0, 2)))
    x48 = xt.reshape(nb, nl, 48)

    freq = jnp.exp(jnp.arange(0, NUM_PE, 2, dtype=jnp.float32)
                   * (-(np.log(10000.0) / NUM_PE)))
    narr = jnp.arange(nl, dtype=jnp.float32)[:, None]
    cn = jnp.cos(narr * freq[None, :])
    sn = jnp.sin(narr * freq[None, :])
    tn1 = jnp.concatenate([cn, sn], axis=1)
    tn2 = jnp.concatenate([sn, cn], axis=1)
    tables = jnp.broadcast_to(
        jnp.concatenate([tn1, tn2], axis=1)[None], (nb, nl, 32))
    x80 = jnp.concatenate([x48, tables], axis=2)
    xhi = x80.astype(jnp.bfloat16)
    xlo = (x80 - xhi.astype(jnp.float32)).astype(jnp.bfloat16)

    uv = jnp.concatenate([cn, cn, sn, -sn], axis=1)

    p = np.arange(NPAIR)
    a_m = jnp.asarray((p[None, :] // N_ATOMS
                       == np.arange(N_ATOMS)[:, None]).astype(np.float32))

    wpe = W_e[:, :NUM_PE].T
    wrbf = (W_e[:, NUM_PE:].T
            .reshape(NPAIR, NUM_RBF, 128)
            .transpose(1, 0, 2))
    gamma = ln_gamma.reshape(1, 128)
    beta = ln_beta.reshape(1, 128)
    return _run(xcaT, xperm, xhi, xlo, a_m, uv, wpe, wrbf, gamma, beta)


# speedup vs baseline: 1.0987x; 1.0026x over previous
import functools
import numpy as np
import jax
import jax.numpy as jnp
from jax.experimental import pallas as pl
from jax.experimental.pallas import tpu as pltpu

NUM_RBF = 16
NUM_PE = 16
TOP_K = 30
N_ATOMS = 14
NPAIR = N_ATOMS * N_ATOMS
LQ = 64


def _body(xcaT_ref, xperm_ref, x80_ref, a_ref, uv_ref,
          wpe_ref, wrbf_ref, g_ref, bt_ref, ekm_ref, eidx_ref, oh_scr):
    t = pl.program_id(1)
    base = t * LQ

    d2 = None
    for c in range(3):
        xall = xcaT_ref[0, c:c + 1, :]
        xq = xperm_ref[0, c, pl.ds(base, LQ), 1:2]
        diff = xq - xall
        d2 = diff * diff if d2 is None else d2 + diff * diff
    dca = jnp.sqrt(d2 + 1e-6)

    lane512 = jax.lax.broadcasted_iota(jnp.int32, (LQ, 512), 1)

    sel_cols = []
    oh_blocks = []
    work = dca
    for k in range(TOP_K):
        m = jnp.min(work, axis=1, keepdims=True)
        cand = jnp.where(work == m, lane512, 512)
        sel = jnp.min(cand, axis=1, keepdims=True)
        hit = lane512 == sel
        work = jnp.where(hit, jnp.float32(np.inf), work)
        sel_cols.append(sel)
        oh_blocks.append(hit.astype(jnp.float32))

    eidx_ref[0] = jnp.concatenate(sel_cols, axis=1)

    oh = jnp.concatenate(oh_blocks, axis=0)
    x80f = x80_ref[0]
    xhi = x80f.astype(jnp.bfloat16)
    xlo = (x80f - xhi.astype(jnp.float32)).astype(jnp.bfloat16)
    xn80 = (jnp.dot(oh, xhi, preferred_element_type=jnp.float32)
            + jnp.dot(oh, xlo, preferred_element_type=jnp.float32))

    urep = jnp.concatenate([uv_ref[:, :NUM_PE]] * TOP_K, axis=0)
    vrep = jnp.concatenate([uv_ref[:, NUM_PE:]] * TOP_K, axis=0)
    epos = xn80[:, 48:64] * urep + xn80[:, 64:80] * vrep

    hp = jax.lax.Precision.HIGHEST
    d2nb = None
    for c in range(3):
        qc = xperm_ref[0, c, pl.ds(base, LQ), :]
        xn = xn80[:, c * 16:c * 16 + N_ATOMS]
        nexp = jnp.concatenate([xn] * N_ATOMS, axis=1)
        qa = jnp.dot(qc, a_ref[...], preferred_element_type=jnp.float32,
                     precision=hp)
        qexp = jnp.concatenate([qa] * TOP_K, axis=0)
        diff = qexp - nexp
        d2nb = diff * diff if d2nb is None else d2nb + diff * diff
    dnb = jnp.sqrt(d2nb + 1e-6)

    acc = jnp.dot(epos, wpe_ref[...], preferred_element_type=jnp.float32)
    mus = np.linspace(0.0, 20.0, NUM_RBF).astype(np.float32)
    inv_sigma = np.float32(NUM_RBF / 20.0)
    u = dnb * inv_sigma
    w = -(u * u)
    for mi in range(NUM_RBF):
        cm = np.float32(mus[mi] * inv_sigma)
        g = jnp.exp(u * np.float32(2.0 * cm) + (w - np.float32(cm * cm)))
        acc = acc + jnp.dot(g, wrbf_ref[mi],
                            preferred_element_type=jnp.float32)

    mu = jnp.mean(acc, axis=1, keepdims=True)
    xc_ = acc - mu
    var = jnp.mean(xc_ * xc_, axis=1, keepdims=True)
    y = (xc_ * jax.lax.rsqrt(var + 1e-5)) * g_ref[0:1, :] + bt_ref[0:1, :]

    ekm_ref[0] = y.reshape(TOP_K, LQ, 128)


@jax.jit
def _run(xcaT, xperm, x80, a_m, uv, wpe, wrbf, gamma, beta):
    nb, _, nl = xcaT.shape
    grid = (nb, nl // LQ)
    ekm, eidx = pl.pallas_call(
        _body,
        grid=grid,
        in_specs=[
            pl.BlockSpec((1, 3, nl), lambda b, t: (b, 0, 0)),
            pl.BlockSpec((1, 3, nl, N_ATOMS), lambda b, t: (b, 0, 0, 0)),
            pl.BlockSpec((1, nl, 80), lambda b, t: (b, 0, 0)),
            pl.BlockSpec((N_ATOMS, NPAIR), lambda b, t: (0, 0)),
            pl.BlockSpec((LQ, 2 * NUM_PE), lambda b, t: (t, 0)),
            pl.BlockSpec((NUM_PE, 128), lambda b, t: (0, 0)),
            pl.BlockSpec((NUM_RBF, NPAIR, 128), lambda b, t: (0, 0, 0)),
            pl.BlockSpec((1, 128), lambda b, t: (0, 0)),
            pl.BlockSpec((1, 128), lambda b, t: (0, 0)),
        ],
        scratch_shapes=[pltpu.VMEM((LQ * TOP_K, 512), jnp.float32)],
        out_specs=[
            pl.BlockSpec((1, TOP_K, LQ, 128), lambda b, t: (b, 0, t, 0)),
            pl.BlockSpec((1, LQ, TOP_K), lambda b, t: (b, t, 0)),
        ],
        out_shape=[
            jax.ShapeDtypeStruct((nb, TOP_K, nl, 128), jnp.float32),
            jax.ShapeDtypeStruct((nb, nl, TOP_K), jnp.int32),
        ],
        compiler_params=pltpu.CompilerParams(
            dimension_semantics=("parallel", "arbitrary"),
        ),
    )(xcaT, xperm, x80, a_m, uv, wpe, wrbf, gamma, beta)
    e_out = jnp.transpose(ekm, (0, 2, 1, 3))
    return e_out, eidx


def kernel(X, mask, atom_mask, W_e, ln_gamma, ln_beta):
    xperm = jnp.transpose(X, (0, 3, 1, 2))
    xcaT = xperm[:, :, :, 1]

    nb, nl = X.shape[0], X.shape[1]
    xt = jnp.transpose(X, (0, 1, 3, 2))
    xt = jnp.pad(xt, ((0, 0), (0, 0), (0, 0), (0, 2)))
    x48 = xt.reshape(nb, nl, 48)

    freq = jnp.exp(jnp.arange(0, NUM_PE, 2, dtype=jnp.float32)
                   * (-(np.log(10000.0) / NUM_PE)))
    narr = jnp.arange(nl, dtype=jnp.float32)[:, None]
    cn = jnp.cos(narr * freq[None, :])
    sn = jnp.sin(narr * freq[None, :])
    tn1 = jnp.concatenate([cn, sn], axis=1)
    tn2 = jnp.concatenate([sn, cn], axis=1)
    tables = jnp.broadcast_to(
        jnp.concatenate([tn1, tn2], axis=1)[None], (nb, nl, 32))
    x80 = jnp.concatenate([x48, tables], axis=2)

    uv = jnp.concatenate([cn, cn, sn, -sn], axis=1)

    p = np.arange(NPAIR)
    a_m = jnp.asarray((p[None, :] // N_ATOMS
                       == np.arange(N_ATOMS)[:, None]).astype(np.float32))

    wpe = W_e[:, :NUM_PE].T
    wrbf = (W_e[:, NUM_PE:].T
            .reshape(NPAIR, NUM_RBF, 128)
            .transpose(1, 0, 2))
    gamma = ln_gamma.reshape(1, 128)
    beta = ln_beta.reshape(1, 128)
    return _run(xcaT, xperm, x80, a_m, uv, wpe, wrbf, gamma, beta)
